# baseline (speedup 1.0000x reference)
import functools

import jax
import jax.numpy as jnp
from jax import lax
from jax.experimental import pallas as pl
from jax.experimental.pallas import tpu as pltpu
from jax.experimental.pallas import tpu_sc as plsc

POS_OFFSET = 2
LANES = 16


@functools.lru_cache(maxsize=None)
def _make_kernel(B, S, V, H, NC, NS):
    NW = NC * NS
    PB = 2
    PAIRS = B // PB
    WPP = NW // PAIRS
    SW = S // WPP
    C = 4
    NCH = SW // C
    H2 = H // 2
    HR = 2 * C
    ROWS = PB * HR
    NIDX = NCH * ROWS
    S2 = 2 * S
    vecs = H2 // LANES
    NBUF = 4

    mesh = plsc.VectorSubcoreMesh(core_axis_name="c", subcore_axis_name="s")

    @functools.partial(
        pl.kernel,
        mesh=mesh,
        out_type=jax.ShapeDtypeStruct((B * S2, H2), jnp.float32),
        scratch_types=[
            pltpu.VMEM((NIDX,), jnp.int32),
            *[pltpu.VMEM((LANES,), jnp.int32) for _ in range(NBUF)],
            *[pltpu.VMEM((ROWS, H2), jnp.float32) for _ in range(NBUF)],
            *[pltpu.VMEM((HR, H2), jnp.float32) for _ in range(NBUF)],
            *[pltpu.SemaphoreType.DMA for _ in range(3 * NBUF)],
        ],
    )
    def emb(ids_hbm, tok_hbm, pos_hbm, out_hbm, idx_v,
            pidx0, pidx1, pidx2, pidx3, tok0, tok1, tok2, tok3,
            pos0, pos1, pos2, pos3,
            ts0, ts1, ts2, ts3, ps0, ps1, ps2, ps3, ss0, ss1, ss2, ss3):
        wid = lax.axis_index("s") * NC + lax.axis_index("c")
        pair = wid // WPP
        s0 = (wid % WPP) * SW
        pltpu.sync_copy(ids_hbm.at[pl.ds(wid * NIDX, NIDX)], idx_v)

        pidx = (pidx0, pidx1, pidx2, pidx3)
        toks = (tok0, tok1, tok2, tok3)
        poss = (pos0, pos1, pos2, pos3)
        tsem = (ts0, ts1, ts2, ts3)
        psem = (ps0, ps1, ps2, ps3)
        ssem = (ss0, ss1, ss2, ss3)

        def tok_desc(g, m):
            return pltpu.make_async_copy(
                tok_hbm.at[idx_v.at[pl.ds(g * ROWS, ROWS)]], toks[m], tsem[m]
            )

        def pos_desc(g, m):
            return pltpu.make_async_copy(
                pos_hbm.at[pidx[m].at[pl.ds(0, HR)]], poss[m], psem[m]
            )

        def st_desc(g, m, b2):
            row = ((pair * PB + b2) * S2 + (s0 + g * C) * 2)
            return pltpu.make_async_copy(
                toks[m].at[pl.ds(b2 * HR, HR)], out_hbm.at[pl.ds(row, HR)],
                ssem[m],
            )

        def issue(g, m):
            tok_desc(g, m).start()
            pidx[m][...] = lax.iota(jnp.int32, LANES) + (
                2 * (s0 + POS_OFFSET + g * C)
            )
            pos_desc(g, m).start()

        issue(0, 0)
        issue(1, 1)
        issue(2, 2)

        def body(i, _):
            for m in range(NBUF):
                g = i * NBUF + m
                tok_b, pos_b = toks[m], poss[m]
                tok_desc(g, m).wait()
                pos_desc(g, m).wait()

                def add_row(hr, _, tok_b=tok_b, pos_b=pos_b):
                    def add_vec(j, _):
                        col = j * LANES
                        pvec = pos_b[hr, pl.ds(col, LANES)]
                        for b2 in range(PB):
                            plsc.addupdate(
                                tok_b.at[b2 * HR + hr, pl.ds(col, LANES)],
                                pvec,
                            )
                        return _
                    return lax.fori_loop(0, vecs, add_vec, _)

                lax.fori_loop(0, HR, add_row, None)

                for b2 in range(PB):
                    st_desc(g, m, b2).start()

                m3 = (m + 3) % NBUF

                @pl.when(g + 3 < NCH)
                def _next(g=g, m3=m3):
                    @pl.when(g >= 1)
                    def _drain(g=g, m3=m3):
                        for b2 in range(PB):
                            st_desc(g - 1, m3, b2).wait()

                    issue(g + 3, m3)

            return _

        lax.fori_loop(0, NCH // NBUF, body, None)

        for gl in range(NCH - NBUF, NCH):
            for b2 in range(PB):
                st_desc(gl, gl % NBUF, b2).wait()

    return emb


def kernel(token_ids, token_table, pos_table):
    B, S = token_ids.shape
    V, H = token_table.shape
    info = plsc.get_sparse_core_info()
    NC, NS = info.num_cores, info.num_subcores
    NW = NC * NS
    PB = 2
    PAIRS = B // PB
    WPP = NW // PAIRS
    SW = S // WPP
    C = 4
    t2 = token_ids * 2
    ids2 = jnp.stack([t2, t2 + 1], axis=-1)
    ids_perm = (
        ids2.reshape(PAIRS, PB, WPP, SW // C, C, 2)
        .transpose(0, 2, 3, 1, 4, 5)
        .reshape(-1)
    )
    emb = _make_kernel(B, S, V, H, NC, NS)
    out = emb(
        ids_perm,
        token_table.reshape(2 * V, H // 2),
        pos_table.reshape(-1, H // 2),
    )
    return out.reshape(B, S, H)


# speedup vs baseline: 7.2748x; 7.2748x over previous
import functools

import jax
import jax.numpy as jnp
from jax import lax
from jax.experimental import pallas as pl
from jax.experimental.pallas import tpu as pltpu
from jax.experimental.pallas import tpu_sc as plsc

POS_OFFSET = 2
LANES = 16


@functools.lru_cache(maxsize=None)
def _make_kernel(B, S, V, H, NC, NS):
    NW = NC * NS
    PB = 2
    PAIRS = B // PB
    WPP = NW // PAIRS
    SW = S // WPP
    C = 8
    NCH = SW // C
    ROWS = PB * C
    rows_per_w = PB * SW
    total_rows = B * S
    vecs = H // LANES

    mesh = plsc.VectorSubcoreMesh(core_axis_name="c", subcore_axis_name="s")

    @functools.partial(
        pl.kernel,
        mesh=mesh,
        out_type=jax.ShapeDtypeStruct((total_rows, H), jnp.float32),
        scratch_types=[
            pltpu.VMEM((rows_per_w,), jnp.int32),
            pltpu.VMEM((LANES,), jnp.int32),
            pltpu.VMEM((LANES,), jnp.int32),
            pltpu.VMEM((ROWS, H), jnp.float32),
            pltpu.VMEM((ROWS, H), jnp.float32),
            pltpu.VMEM((C, H), jnp.float32),
            pltpu.VMEM((C, H), jnp.float32),
            pltpu.VMEM((C, H), jnp.float32),
            pltpu.SemaphoreType.DMA,
            pltpu.SemaphoreType.DMA,
            pltpu.SemaphoreType.DMA,
            pltpu.SemaphoreType.DMA,
            pltpu.SemaphoreType.DMA,
            pltpu.SemaphoreType.DMA,
        ],
    )
    def emb(ids_hbm, tok_hbm, pos_hbm, out_hbm, idx_v, pidx0, pidx1,
            tok0, tok1, pos0, pos1, stage,
            ts0, ts1, ps0, ps1, s0sem, s1sem):
        wid = lax.axis_index("s") * NC + lax.axis_index("c")
        pair = wid // WPP
        s0 = (wid % WPP) * SW
        pltpu.sync_copy(ids_hbm.at[pl.ds(wid * rows_per_w, rows_per_w)], idx_v)

        pidx = (pidx0, pidx1)
        toks = (tok0, tok1)
        poss = (pos0, pos1)
        tsem = (ts0, ts1)
        psem = (ps0, ps1)

        def tok_desc(g, m):
            return pltpu.make_async_copy(
                tok_hbm.at[idx_v.at[pl.ds(g * ROWS, ROWS)]], toks[m], tsem[m]
            )

        def pos_desc(g, m):
            return pltpu.make_async_copy(
                pos_hbm.at[pidx[m].at[pl.ds(0, C)]], poss[m], psem[m]
            )

        def st0_desc(g, m):
            row = pair * PB * S + s0 + g * C
            return pltpu.make_async_copy(
                toks[m].at[pl.ds(0, C)], out_hbm.at[pl.ds(row, C)], s0sem
            )

        def st1_desc(g):
            row = (pair * PB + 1) * S + s0 + g * C
            return pltpu.make_async_copy(
                stage, out_hbm.at[pl.ds(row, C)], s1sem
            )

        def issue(g, m):
            tok_desc(g, m).start()
            pidx[m][...] = lax.iota(jnp.int32, LANES) + (s0 + POS_OFFSET + g * C)
            pos_desc(g, m).start()

        issue(0, 0)
        issue(1, 1)

        def body(i, _):
            for m in range(2):
                g = 2 * i + m
                tok_b, pos_b = toks[m], poss[m]
                tok_desc(g, m).wait()
                pos_desc(g, m).wait()

                @plsc.parallel_loop(0, vecs)
                def add0(j, tok_b=tok_b, pos_b=pos_b):
                    col = j * LANES
                    for hr in range(C):
                        plsc.addupdate(
                            tok_b.at[hr, pl.ds(col, LANES)],
                            pos_b[hr, pl.ds(col, LANES)],
                        )

                st0_desc(g, m).start()

                @pl.when(g >= 1)
                def _drain1(g=g):
                    st1_desc(g - 1).wait()

                @plsc.parallel_loop(0, vecs)
                def add1(j, tok_b=tok_b, pos_b=pos_b):
                    col = j * LANES
                    for hr in range(C):
                        stage[hr, pl.ds(col, LANES)] = (
                            tok_b[C + hr, pl.ds(col, LANES)]
                            + pos_b[hr, pl.ds(col, LANES)]
                        )

                st1_desc(g).start()

                st0_desc(g, m).wait()

                @pl.when(g + 2 < NCH)
                def _next(g=g, m=m):
                    issue(g + 2, m)

            return _

        lax.fori_loop(0, NCH // 2, body, None)
        st1_desc(NCH - 1).wait()

    return emb


def kernel(token_ids, token_table, pos_table):
    B, S = token_ids.shape
    V, H = token_table.shape
    info = plsc.get_sparse_core_info()
    NC, NS = info.num_cores, info.num_subcores
    NW = NC * NS
    PB = 2
    PAIRS = B // PB
    WPP = NW // PAIRS
    SW = S // WPP
    C = 8
    ids_perm = (
        token_ids.reshape(PAIRS, PB, WPP, SW // C, C)
        .transpose(0, 2, 3, 1, 4)
        .reshape(B * S)
    )
    emb = _make_kernel(B, S, V, H, NC, NS)
    out = emb(ids_perm, token_table, pos_table)
    return out.reshape(B, S, H)


# speedup vs baseline: 7.3527x; 1.0107x over previous
import functools

import jax
import jax.numpy as jnp
from jax import lax
from jax.experimental import pallas as pl
from jax.experimental.pallas import tpu as pltpu
from jax.experimental.pallas import tpu_sc as plsc

POS_OFFSET = 2
LANES = 16


@functools.lru_cache(maxsize=None)
def _make_kernel(B, S, V, H, NC, NS):
    NW = NC * NS
    PB = 2
    PAIRS = B // PB
    WPP = NW // PAIRS
    SW = S // WPP
    C = 8
    NCH = SW // C
    ROWS = PB * C
    rows_per_w = PB * SW
    total_rows = B * S
    vecs = H // LANES

    mesh = plsc.VectorSubcoreMesh(core_axis_name="c", subcore_axis_name="s")

    @functools.partial(
        pl.kernel,
        mesh=mesh,
        out_type=jax.ShapeDtypeStruct((total_rows, H), jnp.float32),
        scratch_types=[
            pltpu.VMEM((rows_per_w,), jnp.int32),
            pltpu.VMEM((LANES,), jnp.int32),
            pltpu.VMEM((LANES,), jnp.int32),
            pltpu.VMEM((ROWS, H), jnp.float32),
            pltpu.VMEM((ROWS, H), jnp.float32),
            pltpu.VMEM((C, H), jnp.float32),
            pltpu.VMEM((C, H), jnp.float32),
            pltpu.VMEM((C, H), jnp.float32),
            pltpu.SemaphoreType.DMA,
            pltpu.SemaphoreType.DMA,
            pltpu.SemaphoreType.DMA,
            pltpu.SemaphoreType.DMA,
            pltpu.SemaphoreType.DMA,
            pltpu.SemaphoreType.DMA,
            pltpu.SemaphoreType.DMA,
            pltpu.SemaphoreType.DMA,
        ],
    )
    def emb(ids_hbm, tok_hbm, pos_hbm, out_hbm, idx_v, pidx0, pidx1,
            tok0, tok1, pos0, pos1, stage,
            ta0, ta1, tb0, tb1, ps0, ps1, s0sem, s1sem):
        wid = lax.axis_index("s") * NC + lax.axis_index("c")
        pair = wid // WPP
        s0 = (wid % WPP) * SW
        pltpu.sync_copy(ids_hbm.at[pl.ds(wid * rows_per_w, rows_per_w)], idx_v)

        pidx = (pidx0, pidx1)
        toks = (tok0, tok1)
        poss = (pos0, pos1)
        tsemA = (ta0, ta1)
        tsemB = (tb0, tb1)
        psem = (ps0, ps1)

        def tokA_desc(g, m):
            return pltpu.make_async_copy(
                tok_hbm.at[idx_v.at[pl.ds(g * ROWS, C)]],
                toks[m].at[pl.ds(0, C)], tsemA[m]
            )

        def tokB_desc(g, m):
            return pltpu.make_async_copy(
                tok_hbm.at[idx_v.at[pl.ds(g * ROWS + C, C)]],
                toks[m].at[pl.ds(C, C)], tsemB[m]
            )

        def pos_desc(g, m):
            return pltpu.make_async_copy(
                pos_hbm.at[pidx[m].at[pl.ds(0, C)]], poss[m], psem[m]
            )

        def st0_desc(g, m):
            row = pair * PB * S + s0 + g * C
            return pltpu.make_async_copy(
                toks[m].at[pl.ds(0, C)], out_hbm.at[pl.ds(row, C)], s0sem
            )

        def st1_desc(g):
            row = (pair * PB + 1) * S + s0 + g * C
            return pltpu.make_async_copy(
                stage, out_hbm.at[pl.ds(row, C)], s1sem
            )

        def issue(g, m):
            tokA_desc(g, m).start()
            tokB_desc(g, m).start()
            pidx[m][...] = lax.iota(jnp.int32, LANES) + (s0 + POS_OFFSET + g * C)
            pos_desc(g, m).start()

        issue(0, 0)
        issue(1, 1)

        def body(i, _):
            for m in range(2):
                g = 2 * i + m
                tok_b, pos_b = toks[m], poss[m]
                tokA_desc(g, m).wait()
                pos_desc(g, m).wait()

                @plsc.parallel_loop(0, vecs)
                def add0(j, tok_b=tok_b, pos_b=pos_b):
                    col = j * LANES
                    for hr in range(C):
                        plsc.addupdate(
                            tok_b.at[hr, pl.ds(col, LANES)],
                            pos_b[hr, pl.ds(col, LANES)],
                        )

                st0_desc(g, m).start()

                @pl.when(g >= 1)
                def _drain1(g=g):
                    st1_desc(g - 1).wait()

                tokB_desc(g, m).wait()

                @plsc.parallel_loop(0, vecs)
                def add1(j, tok_b=tok_b, pos_b=pos_b):
                    col = j * LANES
                    for hr in range(C):
                        stage[hr, pl.ds(col, LANES)] = (
                            tok_b[C + hr, pl.ds(col, LANES)]
                            + pos_b[hr, pl.ds(col, LANES)]
                        )

                st1_desc(g).start()

                st0_desc(g, m).wait()

                @pl.when(g + 2 < NCH)
                def _next(g=g, m=m):
                    issue(g + 2, m)

            return _

        lax.fori_loop(0, NCH // 2, body, None)
        st1_desc(NCH - 1).wait()

    return emb


def kernel(token_ids, token_table, pos_table):
    B, S = token_ids.shape
    V, H = token_table.shape
    info = plsc.get_sparse_core_info()
    NC, NS = info.num_cores, info.num_subcores
    NW = NC * NS
    PB = 2
    PAIRS = B // PB
    WPP = NW // PAIRS
    SW = S // WPP
    C = 8
    ids_perm = (
        token_ids.reshape(PAIRS, PB, WPP, SW // C, C)
        .transpose(0, 2, 3, 1, 4)
        .reshape(B * S)
    )
    emb = _make_kernel(B, S, V, H, NC, NS)
    out = emb(ids_perm, token_table, pos_table)
    return out.reshape(B, S, H)


# speedup vs baseline: 7.7460x; 1.0535x over previous
import functools

import jax
import jax.numpy as jnp
from jax import lax
from jax.experimental import pallas as pl
from jax.experimental.pallas import tpu as pltpu
from jax.experimental.pallas import tpu_sc as plsc

POS_OFFSET = 2
LANES = 16


@functools.lru_cache(maxsize=None)
def _make_kernel(B, S, V, H, NC, NS):
    NW = NC * NS
    SUBS = B // 2
    SW = S // NW
    C = 8
    NCH = SW // C
    ROWS = 2 * C
    NIDX = NCH * SUBS * ROWS
    total_rows = B * S
    vecs = H // LANES

    mesh = plsc.VectorSubcoreMesh(core_axis_name="c", subcore_axis_name="s")

    @functools.partial(
        pl.kernel,
        mesh=mesh,
        out_type=jax.ShapeDtypeStruct((total_rows, H), jnp.float32),
        scratch_types=[
            pltpu.VMEM((NIDX,), jnp.int32),
            pltpu.VMEM((LANES,), jnp.int32),
            pltpu.VMEM((LANES,), jnp.int32),
            pltpu.VMEM((ROWS, H), jnp.float32),
            pltpu.VMEM((ROWS, H), jnp.float32),
            pltpu.VMEM((C, H), jnp.float32),
            pltpu.VMEM((C, H), jnp.float32),
            pltpu.VMEM((C, H), jnp.float32),
            pltpu.SemaphoreType.DMA,
            pltpu.SemaphoreType.DMA,
            pltpu.SemaphoreType.DMA,
            pltpu.SemaphoreType.DMA,
            pltpu.SemaphoreType.DMA,
            pltpu.SemaphoreType.DMA,
            pltpu.SemaphoreType.DMA,
            pltpu.SemaphoreType.DMA,
        ],
    )
    def emb(ids_hbm, tok_hbm, pos_hbm, out_hbm, idx_v, pidx0, pidx1,
            tok0, tok1, pos0, pos1, stage,
            ta0, ta1, tb0, tb1, ps0, ps1, s0sem, s1sem):
        wid = lax.axis_index("s") * NC + lax.axis_index("c")
        s0 = wid * SW
        pltpu.sync_copy(ids_hbm.at[pl.ds(wid * NIDX, NIDX)], idx_v)

        pidx = (pidx0, pidx1)
        toks = (tok0, tok1)
        poss = (pos0, pos1)
        tsemA = (ta0, ta1)
        tsemB = (tb0, tb1)
        psem = (ps0, ps1)

        def tokA_desc(c, sub):
            off = (c * SUBS + sub) * ROWS
            return pltpu.make_async_copy(
                tok_hbm.at[idx_v.at[pl.ds(off, C)]],
                toks[sub].at[pl.ds(0, C)], tsemA[sub]
            )

        def tokB_desc(c, sub):
            off = (c * SUBS + sub) * ROWS + C
            return pltpu.make_async_copy(
                tok_hbm.at[idx_v.at[pl.ds(off, C)]],
                toks[sub].at[pl.ds(C, C)], tsemB[sub]
            )

        def pos_desc(c, m):
            return pltpu.make_async_copy(
                pos_hbm.at[pidx[m].at[pl.ds(0, C)]], poss[m], psem[m]
            )

        def st0_desc(c, sub):
            row = (2 * sub) * S + s0 + c * C
            return pltpu.make_async_copy(
                toks[sub].at[pl.ds(0, C)], out_hbm.at[pl.ds(row, C)], s0sem
            )

        def st1_desc(c, sub):
            row = (2 * sub + 1) * S + s0 + c * C
            return pltpu.make_async_copy(
                stage, out_hbm.at[pl.ds(row, C)], s1sem
            )

        def issue_tok(c, sub):
            tokA_desc(c, sub).start()
            tokB_desc(c, sub).start()

        def issue_pos(c, m):
            pidx[m][...] = lax.iota(jnp.int32, LANES) + (s0 + POS_OFFSET + c * C)
            pos_desc(c, m).start()

        def process(c, sub, pos_b, first):
            tok_b = toks[sub]
            tokA_desc(c, sub).wait()

            @plsc.parallel_loop(0, vecs)
            def add0(j):
                col = j * LANES
                for hr in range(C):
                    plsc.addupdate(
                        tok_b.at[hr, pl.ds(col, LANES)],
                        pos_b[hr, pl.ds(col, LANES)],
                    )

            st0_desc(c, sub).start()

            if first:
                @pl.when(c >= 1)
                def _drain1():
                    st1_desc(c, sub).wait()
            else:
                st1_desc(c, sub).wait()

            tokB_desc(c, sub).wait()

            @plsc.parallel_loop(0, vecs)
            def add1(j):
                col = j * LANES
                for hr in range(C):
                    stage[hr, pl.ds(col, LANES)] = (
                        tok_b[C + hr, pl.ds(col, LANES)]
                        + pos_b[hr, pl.ds(col, LANES)]
                    )

            st1_desc(c, sub).start()

            st0_desc(c, sub).wait()

        issue_pos(0, 0)
        issue_pos(1, 1)
        issue_tok(0, 0)
        issue_tok(0, 1)

        def body(cc, _):
            for par in range(2):
                c = 2 * cc + par
                pos_b = poss[par]
                pos_desc(c, par).wait()
                if par == 0:
                    issue_pos(c + 1, 1)
                else:
                    @pl.when(cc + 1 < NCH // 2)
                    def _pos_next(c=c):
                        issue_pos(c + 1, 0)

                for sub in range(SUBS):
                    process(c, sub, pos_b, first=(par == 0 and sub == 0))
                    if par == 0:
                        issue_tok(c + 1, sub)
                    else:
                        @pl.when(cc + 1 < NCH // 2)
                        def _tok_next(c=c, sub=sub):
                            issue_tok(c + 1, sub)

            return _

        lax.fori_loop(0, NCH // 2, body, None)
        st1_desc(NCH - 1, SUBS - 1).wait()

    return emb


def kernel(token_ids, token_table, pos_table):
    B, S = token_ids.shape
    V, H = token_table.shape
    info = plsc.get_sparse_core_info()
    NC, NS = info.num_cores, info.num_subcores
    NW = NC * NS
    SW = S // NW
    C = 8
    ids_perm = (
        token_ids.reshape(B // 2, 2, NW, SW // C, C)
        .transpose(2, 3, 0, 1, 4)
        .reshape(B * S)
    )
    emb = _make_kernel(B, S, V, H, NC, NS)
    out = emb(ids_perm, token_table, pos_table)
    return out.reshape(B, S, H)
